# baseline (speedup 1.0000x reference)
import functools

import jax
import jax.numpy as jnp
from jax import lax
from jax.experimental import pallas as pl
from jax.experimental.pallas import tpu as pltpu
from jax.experimental.pallas import tpu_sc as plsc

B, L, H = 4, 2048, 1024
N = B * L
NC, NS = 2, 16
NW = NC * NS
P = L // NW
PC = 8
NCH = P // PC
CR = B * PC
RING = 3
PRING = 3
AHEAD = 2
LANES = 16


def kernel(X, W_word, W_pos):
    idx = (X.reshape(B, NW, NCH, PC)
            .transpose(1, 2, 0, 3)
            .reshape(N)
            .astype(jnp.int32))
    mesh = plsc.VectorSubcoreMesh(core_axis_name="c", subcore_axis_name="s")

    @functools.partial(
        pl.kernel,
        out_type=jax.ShapeDtypeStruct((N, H), jnp.float32),
        mesh=mesh,
        scratch_types=(
            [pltpu.VMEM((B * P,), jnp.int32)]
            + [pltpu.VMEM((CR, H), jnp.float32) for _ in range(RING)]
            + [pltpu.VMEM((PC, H), jnp.float32) for _ in range(PRING)]
            + [pltpu.SemaphoreType.DMA for _ in range(2 * RING + PRING)]
        ),
    )
    def embed(w_hbm, p_hbm, i_hbm, o_hbm, *scr):
        idx_v = scr[0]
        ring = list(scr[1:1 + RING])
        pos = list(scr[1 + RING:1 + RING + PRING])
        sems = list(scr[1 + RING + PRING:])
        sg = sems[:RING]
        so = sems[RING:2 * RING]
        sp = sems[2 * RING:]

        wid = lax.axis_index("s") * NC + lax.axis_index("c")
        l0 = wid * P

        pltpu.sync_copy(i_hbm.at[pl.ds(wid * (B * P), B * P)], idx_v)

        def start(k):
            p = k % RING
            g = pltpu.async_copy(
                w_hbm.at[idx_v.at[pl.ds(k * CR, CR)]], ring[p], sg[p])
            q = pltpu.async_copy(
                p_hbm.at[pl.ds(l0 + k * PC, PC)], pos[k % PRING],
                sp[k % PRING])
            return g, q

        inflight = {k: start(k) for k in range(AHEAD)}
        out_cp = {}

        for k in range(NCH):
            p = k % RING
            g, q = inflight.pop(k)
            g.wait()
            q.wait()

            if k + AHEAD < NCH:
                kd = k + AHEAD - RING
                if kd >= 0:
                    for b in range(B):
                        out_cp.pop((kd, b)).wait()
                inflight[k + AHEAD] = start(k + AHEAD)

            @pl.loop(0, PC)
            def _(r):
                @plsc.parallel_loop(0, H, step=LANES, unroll=8)
                def _(c):
                    s = pl.ds(c, LANES)
                    pv = pos[k % PRING].at[r, s][...]
                    for b in range(B):
                        ring[p].at[b * PC + r, s][...] = jnp.maximum(
                            ring[p].at[b * PC + r, s][...] + pv, 0.0)

            for b in range(B):
                out_cp[(k, b)] = pltpu.async_copy(
                    ring[p].at[pl.ds(b * PC, PC)],
                    o_hbm.at[pl.ds(b * L + l0 + k * PC, PC)],
                    so[p])

        for kk in list(out_cp.keys()):
            out_cp.pop(kk).wait()

    out = embed(W_word, W_pos, idx)
    return out.reshape(B, L, H)


# speedup vs baseline: 1.0034x; 1.0034x over previous
import dataclasses
import functools

import jax
import jax.numpy as jnp
from jax import lax
from jax.experimental import pallas as pl
from jax.experimental.pallas import tpu as pltpu
from jax.experimental.pallas import tpu_sc as plsc

B, L, H = 4, 2048, 1024
N = B * L
NC, NS = 2, 16
NW = NC * NS
P = L // NW
PC = 8
NCH = P // PC
CR = B * PC
RING = 3
PRING = 3
AHEAD = 2
LANES = 16


def kernel(X, W_word, W_pos):
    X = X.astype(jnp.int32)
    mesh = plsc.VectorSubcoreMesh(core_axis_name="c", subcore_axis_name="s")

    cp = pltpu.CompilerParams()
    if "needs_layout_passes" in pltpu.CompilerParams.__dataclass_fields__:
        cp = dataclasses.replace(cp, needs_layout_passes=False)

    @functools.partial(
        pl.kernel,
        out_type=jax.ShapeDtypeStruct((N, H), jnp.float32),
        mesh=mesh,
        compiler_params=cp,
        scratch_types=(
            [pltpu.VMEM((B * P,), jnp.int32),
             pltpu.VMEM((B * P,), jnp.int32)]
            + [pltpu.VMEM((CR, H), jnp.float32) for _ in range(RING)]
            + [pltpu.VMEM((PC, H), jnp.float32) for _ in range(PRING)]
            + [pltpu.SemaphoreType.DMA for _ in range(2 * RING + PRING + 1)]
        ),
    )
    def embed(x_hbm, w_hbm, p_hbm, o_hbm, *scr):
        idx_b = scr[0]
        idx_c = scr[1]
        ring = list(scr[2:2 + RING])
        pos = list(scr[2 + RING:2 + RING + PRING])
        sems = list(scr[2 + RING + PRING:])
        sg = sems[:RING]
        so = sems[RING:2 * RING]
        sp = sems[2 * RING:2 * RING + PRING]
        si = sems[2 * RING + PRING]

        wid = lax.axis_index("s") * NC + lax.axis_index("c")
        l0 = wid * P

        icp = [
            pltpu.async_copy(x_hbm.at[b, pl.ds(l0, P)],
                             idx_b.at[pl.ds(b * P, P)], si)
            for b in range(B)
        ]
        for c in icp:
            c.wait()

        lane = lax.iota(jnp.int32, 16)
        j = lane & (PC - 1)
        bh = lane >> 3
        for t in range(B * P // 16):
            k, half = t // 2, t % 2
            src = (2 * half + bh) * P + k * PC + j
            idx_c.at[pl.ds(t * 16, 16)][...] = plsc.load_gather(idx_b, [src])

        def start(k):
            p = k % RING
            g = pltpu.async_copy(
                w_hbm.at[idx_c.at[pl.ds(k * CR, CR)]], ring[p], sg[p])
            q = pltpu.async_copy(
                p_hbm.at[pl.ds(l0 + k * PC, PC)], pos[k % PRING],
                sp[k % PRING])
            return g, q

        inflight = {k: start(k) for k in range(AHEAD)}
        out_cp = {}

        for k in range(NCH):
            p = k % RING
            g, q = inflight.pop(k)
            g.wait()
            q.wait()

            if k + AHEAD < NCH:
                kd = k + AHEAD - RING
                if kd >= 0:
                    for b in range(B):
                        out_cp.pop((kd, b)).wait()
                inflight[k + AHEAD] = start(k + AHEAD)

            @pl.loop(0, PC)
            def _(r):
                @plsc.parallel_loop(0, H, step=LANES, unroll=8)
                def _(c):
                    s = pl.ds(c, LANES)
                    pv = pos[k % PRING].at[r, s][...]
                    for b in range(B):
                        ring[p].at[b * PC + r, s][...] = jnp.maximum(
                            ring[p].at[b * PC + r, s][...] + pv, 0.0)

            for b in range(B):
                out_cp[(k, b)] = pltpu.async_copy(
                    ring[p].at[pl.ds(b * PC, PC)],
                    o_hbm.at[pl.ds(b * L + l0 + k * PC, PC)],
                    so[p])

        for kk in list(out_cp.keys()):
            out_cp.pop(kk).wait()

    out = embed(X, W_word, W_pos)
    return out.reshape(B, L, H)
